# Initial kernel scaffold; baseline (speedup 1.0000x reference)
import jax
import jax.numpy as jnp
from jax.experimental import pallas as pl


def kernel(input_ids, table):
    raise NotImplementedError("write your pallas kernel here")


# speedup vs baseline: 1.6939x; 1.6939x over previous
import functools

import jax
import jax.numpy as jnp
from jax import lax
from jax.experimental import pallas as pl
from jax.experimental.pallas import tpu as pltpu
from jax.experimental.pallas import tpu_sc as plsc

_CHUNK = 64
_LANES = 16


@functools.cache
def _make_gather(B, D, V):
    info = plsc.get_sparse_core_info()
    NC, NS = info.num_cores, info.num_subcores
    NW = NC * NS
    b_per_w = B // NW
    n_chunks = b_per_w // _CHUNK
    mesh = plsc.VectorSubcoreMesh(core_axis_name="c", subcore_axis_name="s")

    @functools.partial(
        pl.kernel,
        mesh=mesh,
        out_type=jax.ShapeDtypeStruct((B, D), jnp.float32),
        scratch_types=[
            pltpu.VMEM((b_per_w,), jnp.int32),
            pltpu.VMEM((_CHUNK, D), jnp.float32),
            pltpu.SemaphoreType.DMA,
        ],
    )
    def k(ids_hbm, table_hbm, out_hbm, idx_v, rows_v, sem):
        wid = lax.axis_index("s") * NC + lax.axis_index("c")
        base = wid * b_per_w
        pltpu.sync_copy(ids_hbm.at[pl.ds(base, b_per_w)], idx_v)

        def clamp_body(i, carry):
            off = pl.multiple_of(i * _LANES, _LANES)
            v = idx_v[pl.ds(off, _LANES)]
            idx_v[pl.ds(off, _LANES)] = jnp.minimum(jnp.maximum(v, 0), V - 1)
            return carry

        lax.fori_loop(0, b_per_w // _LANES, clamp_body, 0)

        def gather_body(g, carry):
            off = pl.multiple_of(g * _CHUNK, _CHUNK)
            pltpu.async_copy(
                table_hbm.at[idx_v.at[pl.ds(off, _CHUNK)]], rows_v, sem
            ).wait()
            pltpu.sync_copy(rows_v, out_hbm.at[pl.ds(base + off, _CHUNK)])
            return carry

        lax.fori_loop(0, n_chunks, gather_body, 0)

    return k


def kernel(input_ids, table):
    bsz, seq = input_ids.shape
    V, D = table.shape
    ids_flat = input_ids.reshape(bsz * seq)
    out = _make_gather(bsz * seq, D, V)(ids_flat, table)
    return out.reshape(bsz, seq, D)


# speedup vs baseline: 1.7871x; 1.0550x over previous
import functools

import jax
import jax.numpy as jnp
from jax import lax
from jax.experimental import pallas as pl
from jax.experimental.pallas import tpu as pltpu
from jax.experimental.pallas import tpu_sc as plsc

_CHUNK = 32
_NBUF = 2
_LANES = 16


@functools.cache
def _make_gather(B, D, V):
    info = plsc.get_sparse_core_info()
    NC, NS = info.num_cores, info.num_subcores
    NW = NC * NS
    b_per_w = B // NW
    n_chunks = b_per_w // _CHUNK
    n_groups = n_chunks // _NBUF
    mesh = plsc.VectorSubcoreMesh(core_axis_name="c", subcore_axis_name="s")

    @functools.partial(
        pl.kernel,
        mesh=mesh,
        out_type=jax.ShapeDtypeStruct((B, D), jnp.float32),
        scratch_types=[
            pltpu.VMEM((b_per_w,), jnp.int32),
            pltpu.VMEM((_NBUF, _CHUNK, D), jnp.float32),
        ]
        + [pltpu.SemaphoreType.DMA] * (2 * _NBUF),
    )
    def k(ids_hbm, table_hbm, out_hbm, idx_v, rows_v, *sems):
        gsems, ssems = sems[:_NBUF], sems[_NBUF:]
        wid = lax.axis_index("s") * NC + lax.axis_index("c")
        base = wid * b_per_w
        pltpu.sync_copy(ids_hbm.at[pl.ds(base, b_per_w)], idx_v)

        def clamp_body(i, carry):
            off = pl.multiple_of(i * _LANES, _LANES)
            v = idx_v[pl.ds(off, _LANES)]
            idx_v[pl.ds(off, _LANES)] = jnp.minimum(jnp.maximum(v, 0), V - 1)
            return carry

        lax.fori_loop(0, b_per_w // _LANES, clamp_body, 0)

        def g_start(c, b):
            off = pl.multiple_of(c * _CHUNK, _CHUNK)
            pltpu.async_copy(
                table_hbm.at[idx_v.at[pl.ds(off, _CHUNK)]], rows_v.at[b], gsems[b]
            )

        def g_wait(b):
            pltpu.make_async_copy(
                table_hbm.at[idx_v.at[pl.ds(0, _CHUNK)]], rows_v.at[b], gsems[b]
            ).wait()

        def s_start(c, b):
            off = pl.multiple_of(c * _CHUNK, _CHUNK)
            pltpu.async_copy(
                rows_v.at[b], out_hbm.at[pl.ds(base + off, _CHUNK)], ssems[b]
            )

        def s_wait(b):
            pltpu.make_async_copy(
                rows_v.at[b], out_hbm.at[pl.ds(base, _CHUNK)], ssems[b]
            ).wait()

        for b in range(_NBUF):
            g_start(b, b)

        def body(p, carry):
            c0 = p * _NBUF
            for b in range(_NBUF):
                g_wait(b)
                s_start(c0 + b, b)
            for b in range(_NBUF):
                s_wait(b)
                g_start(c0 + _NBUF + b, b)
            return carry

        lax.fori_loop(0, n_groups - 1, body, 0)

        c0 = (n_groups - 1) * _NBUF
        for b in range(_NBUF):
            g_wait(b)
            s_start(c0 + b, b)
        for b in range(_NBUF):
            s_wait(b)

    return k


def kernel(input_ids, table):
    bsz, seq = input_ids.shape
    V, D = table.shape
    ids_flat = input_ids.reshape(bsz * seq)
    out = _make_gather(bsz * seq, D, V)(ids_flat, table)
    return out.reshape(bsz, seq, D)


# speedup vs baseline: 1.8236x; 1.0204x over previous
import functools

import jax
import jax.numpy as jnp
from jax import lax
from jax.experimental import pallas as pl
from jax.experimental.pallas import tpu as pltpu
from jax.experimental.pallas import tpu_sc as plsc

_CHUNK = 16
_NBUF = 4
_LANES = 16


@functools.cache
def _make_gather(B, D, V):
    info = plsc.get_sparse_core_info()
    NC, NS = info.num_cores, info.num_subcores
    NW = NC * NS
    b_per_w = B // NW
    n_chunks = b_per_w // _CHUNK
    n_groups = n_chunks // _NBUF
    mesh = plsc.VectorSubcoreMesh(core_axis_name="c", subcore_axis_name="s")

    @functools.partial(
        pl.kernel,
        mesh=mesh,
        out_type=jax.ShapeDtypeStruct((B, D), jnp.float32),
        scratch_types=[
            pltpu.VMEM((b_per_w,), jnp.int32),
            pltpu.VMEM((_NBUF, _CHUNK, D), jnp.float32),
        ]
        + [pltpu.SemaphoreType.DMA] * (2 * _NBUF),
    )
    def k(ids_hbm, table_hbm, out_hbm, idx_v, rows_v, *sems):
        gsems, ssems = sems[:_NBUF], sems[_NBUF:]
        wid = lax.axis_index("s") * NC + lax.axis_index("c")
        base = wid * b_per_w
        pltpu.sync_copy(ids_hbm.at[pl.ds(base, b_per_w)], idx_v)

        def clamp_body(i, carry):
            off = pl.multiple_of(i * _LANES, _LANES)
            v = idx_v[pl.ds(off, _LANES)]
            idx_v[pl.ds(off, _LANES)] = jnp.minimum(jnp.maximum(v, 0), V - 1)
            return carry

        lax.fori_loop(0, b_per_w // _LANES, clamp_body, 0)

        def g_start(c, b):
            off = pl.multiple_of(c * _CHUNK, _CHUNK)
            pltpu.async_copy(
                table_hbm.at[idx_v.at[pl.ds(off, _CHUNK)]], rows_v.at[b], gsems[b]
            )

        def g_wait(b):
            pltpu.make_async_copy(
                table_hbm.at[idx_v.at[pl.ds(0, _CHUNK)]], rows_v.at[b], gsems[b]
            ).wait()

        def s_start(c, b):
            off = pl.multiple_of(c * _CHUNK, _CHUNK)
            pltpu.async_copy(
                rows_v.at[b], out_hbm.at[pl.ds(base + off, _CHUNK)], ssems[b]
            )

        def s_wait(b):
            pltpu.make_async_copy(
                rows_v.at[b], out_hbm.at[pl.ds(base, _CHUNK)], ssems[b]
            ).wait()

        for b in range(_NBUF):
            g_start(b, b)

        def body(p, carry):
            c0 = p * _NBUF
            for b in range(_NBUF):
                g_wait(b)
                s_start(c0 + b, b)
            for b in range(_NBUF):
                s_wait(b)
                g_start(c0 + _NBUF + b, b)
            return carry

        lax.fori_loop(0, n_groups - 1, body, 0)

        c0 = (n_groups - 1) * _NBUF
        for b in range(_NBUF):
            g_wait(b)
            s_start(c0 + b, b)
        for b in range(_NBUF):
            s_wait(b)

    return k


def kernel(input_ids, table):
    bsz, seq = input_ids.shape
    V, D = table.shape
    ids_flat = input_ids.reshape(bsz * seq)
    out = _make_gather(bsz * seq, D, V)(ids_flat, table)
    return out.reshape(bsz, seq, D)
